# baseline (speedup 1.0000x reference)
import functools

import jax
import jax.numpy as jnp
from jax import lax
from jax.experimental import pallas as pl
from jax.experimental.pallas import tpu as pltpu
from jax.experimental.pallas import tpu_sc as plsc

K_NEIGH = 24
EPS = 1e-05

_NC = 2
_NS = 16
_NW = _NC * _NS
_L = 16
_NEG = float("-inf")


def _dist_tau_body(xq_ref, xk_ref, d_ref, tau_ref, *, R, N):
    xq = xq_ref[0]
    xk = xk_ref[0]
    mm = jnp.dot(xq, xk, preferred_element_type=jnp.float32)
    xxa = jnp.sum(xk * xk, axis=0, keepdims=True)
    xxr = jnp.sum(xq * xq, axis=1, keepdims=True)
    d = (2.0 * mm - xxa) - xxr

    cm = jnp.maximum(d[:, :N // 2], d[:, N // 2:])
    w = N // 2
    while w > 128:
        cm = jnp.maximum(cm[:, :w // 2], cm[:, w // 2:])
        w //= 2

    def rnd(_, carry):
        cmc, _m = carry
        m = jnp.max(cmc, axis=1, keepdims=True)
        cmc = jnp.where(cmc == m, _NEG, cmc)
        return cmc, m

    _, tau = lax.fori_loop(0, K_NEIGH, rnd, (cm, cm[:, :1]))

    d_ref[...] = d.reshape(R * 16, 128)
    tau_ref[...] = tau.reshape(R // 128, 128)


def _dist_tau_kernel(B, N, R):
    BN = B * N
    nr = N // R
    return pl.pallas_call(
        functools.partial(_dist_tau_body, R=R, N=N),
        grid=(B, nr),
        in_specs=[
            pl.BlockSpec((1, R, 8), lambda b, r: (b, r, 0)),
            pl.BlockSpec((1, 8, N), lambda b, r: (b, 0, 0)),
        ],
        out_specs=[
            pl.BlockSpec((R * 16, 128), lambda b, r: (b * nr + r, 0)),
            pl.BlockSpec((R // 128, 128), lambda b, r: (b * nr + r, 0)),
        ],
        out_shape=[
            jax.ShapeDtypeStruct((BN * 16, 128), jnp.float32),
            jax.ShapeDtypeStruct((BN // 128, 128), jnp.float32),
        ],
    )


def _dyn_gather16(vec, idx):
    return lax.gather(
        vec, idx[:, None],
        dimension_numbers=lax.GatherDimensionNumbers(
            offset_dims=(), collapsed_slice_dims=(0,), start_index_map=(0,)),
        slice_sizes=(1,),
        mode=lax.GatherScatterMode.PROMISE_IN_BOUNDS)


def _sort16kv(k, v):
    return plsc.sort_key_val(k, v, descending=True)


def _merge2kv(ka, va, kb, vb):
    rkb = lax.rev(kb, (0,))
    rvb = lax.rev(vb, (0,))
    sel = ka >= rkb
    hk, hv = _sort16kv(jnp.where(sel, ka, rkb), jnp.where(sel, va, rvb))
    lk, lv = _sort16kv(jnp.where(sel, rkb, ka), jnp.where(sel, rvb, va))
    return hk, hv, lk, lv


def _select_kernel(BN, N):
    RPW = BN // _NW
    RB = 8
    NB = RPW // RB
    NCHK = N // _L
    SROW = N // 128
    mesh = plsc.VectorSubcoreMesh(core_axis_name="c", subcore_axis_name="s")

    @functools.partial(
        pl.kernel,
        mesh=mesh,
        compiler_params=pltpu.CompilerParams(needs_layout_passes=False),
        out_type=jax.ShapeDtypeStruct((BN * K_NEIGH,), jnp.int32),

        scratch_types=[
            pltpu.VMEM((RB * SROW * 128,), jnp.float32),
            pltpu.VMEM((RPW,), jnp.float32),
            pltpu.VMEM((N + _L,), jnp.int32),
            pltpu.VMEM((RB * K_NEIGH + 2 * _L,), jnp.int32),
        ],
    )
    def k(d_hbm, tau_hbm, idx_hbm, dbuf, tau_v, sidx, outv):
        wid = lax.axis_index("s") * _NC + lax.axis_index("c")
        wrow0 = wid * RPW
        pltpu.sync_copy(tau_hbm.at[pl.ds(wid * RPW, RPW)], tau_v)
        lane = lax.iota(jnp.int32, _L)

        def block_body(g):
            rowbase = wrow0 + g * RB
            pltpu.sync_copy(d_hbm.at[pl.ds(rowbase * SROW * 128,
                                           RB * SROW * 128)], dbuf)

            def row_body(j):
                lr = g * RB + j
                rg = wrow0 + lr
                bt = plsc.load_gather(tau_v,
                                      [jnp.full((_L,), lr, jnp.int32)])

                def sub_body(sb, offv):
                    base_f = (j * SROW + sb) * 128
                    ivb = jnp.full((_L,), sb * 128, jnp.int32)
                    for kk in range(8):
                        v = dbuf[pl.ds(base_f + kk * _L, _L)]
                        m = v >= bt
                        iv = ivb + (kk * _L + lane)
                        pos = (offv + plsc.cumsum(m.astype(jnp.int32))) - 1
                        plsc.store_scatter(sidx, [pos], iv, mask=m)
                        offv = offv + plsc.all_reduce_population_count(m)
                    return offv

                offv = lax.fori_loop(0, SROW, sub_body,
                                     jnp.zeros((_L,), jnp.int32))
                cnt = offv[0]
                sv = (cnt + (_L - 1)) // _L

                def gathered(j2):
                    si = sidx[pl.ds(j2 * _L, _L)]
                    valid = (j2 * _L + lane) < cnt
                    x = plsc.load_gather(
                        dbuf, [si + jnp.full((_L,), j * N, jnp.int32)],
                        mask=valid)
                    return si, jnp.where(valid, x, _NEG)

                s0, x0 = gathered(0)
                k1, i1 = _sort16kv(x0, s0)
                s1, x1 = gathered(1)
                k2, i2 = _sort16kv(x1, s1)

                def tour_body(j2, T):
                    t1k, t1v, t2k, t2v = T
                    sj, xj = gathered(j2)
                    xs, xi = _sort16kv(xj, sj)
                    h1k, h1v, l1k, l1v = _merge2kv(t1k, t1v, xs, xi)
                    h2k, h2v, _lk, _lv = _merge2kv(t2k, t2v, l1k, l1v)
                    return h1k, h1v, h2k, h2v

                t1k, t1v, t2k, t2v = lax.fori_loop(
                    2, sv, tour_body, (k1, i1, k2, i2))
                _fk, f1v, _gk, f2v = _merge2kv(t1k, t1v, t2k, t2v)

                gbv = jnp.full((_L,), (rg // N) * N, jnp.int32)
                ebase = j * K_NEIGH
                outv[pl.ds(ebase, _L)] = f1v + gbv
                outv[pl.ds(ebase + _L, _L)] = f2v + gbv

            pl.loop(0, RB)(row_body)
            pltpu.sync_copy(
                outv.at[pl.ds(0, RB * K_NEIGH)],
                idx_hbm.at[pl.ds(rowbase * K_NEIGH, RB * K_NEIGH)])

        pl.loop(0, NB)(block_body)

    return k


def _gather_reduce_kernel(BN, C, P):
    pts_per_w = BN // _NW
    n_chunks = pts_per_w // P
    CV = C // _L
    mesh = plsc.VectorSubcoreMesh(core_axis_name="c", subcore_axis_name="s")

    @functools.partial(
        pl.kernel,
        mesh=mesh,
        compiler_params=pltpu.CompilerParams(needs_layout_passes=False),
        out_type=[
            jax.ShapeDtypeStruct((BN, C), jnp.float32),
            jax.ShapeDtypeStruct((BN, C), jnp.float32),
            jax.ShapeDtypeStruct((_NW, _L), jnp.float32),
        ],
        scratch_types=[
            pltpu.VMEM((P * K_NEIGH,), jnp.int32),
            pltpu.VMEM((P * K_NEIGH, C), jnp.float32),
            pltpu.VMEM((P, C), jnp.float32),
            pltpu.VMEM((P, C), jnp.float32),
            pltpu.VMEM((P, C), jnp.float32),
            pltpu.VMEM((_L,), jnp.float32),
            pltpu.SemaphoreType.DMA,
        ],
    )
    def k(feats_hbm, gidx_hbm, omax_hbm, omin_hbm, ss_hbm,
          idx_v, rows_v, cen_v, mxb_v, mnb_v, ss_v, sem):
        wid = lax.axis_index("s") * _NC + lax.axis_index("c")
        w_base = wid * pts_per_w
        ss_v[...] = jnp.zeros((_L,), jnp.float32)

        def chunk_body(t):
            base = w_base + t * P
            pltpu.sync_copy(gidx_hbm.at[pl.ds(base * K_NEIGH, P * K_NEIGH)],
                            idx_v)
            pltpu.async_copy(feats_hbm.at[idx_v], rows_v, sem).wait()
            pltpu.sync_copy(feats_hbm.at[pl.ds(base, P)], cen_v)

            def pt_body(p):
                ssa = ss_v[...]
                for c8 in range(CV):
                    cen = cen_v[p, pl.ds(c8 * _L, _L)]
                    d0 = rows_v[p * K_NEIGH, pl.ds(c8 * _L, _L)] - cen
                    mx = d0
                    mn = d0
                    sq = d0 * d0
                    for kk in range(1, K_NEIGH):
                        d = rows_v[p * K_NEIGH + kk, pl.ds(c8 * _L, _L)] - cen
                        mx = jnp.maximum(mx, d)
                        mn = jnp.minimum(mn, d)
                        sq = sq + d * d
                    mxb_v[p, pl.ds(c8 * _L, _L)] = mx
                    mnb_v[p, pl.ds(c8 * _L, _L)] = mn
                    ssa = ssa + sq
                ss_v[...] = ssa

            pl.loop(0, P)(pt_body)
            pltpu.sync_copy(mxb_v, omax_hbm.at[pl.ds(base, P)])
            pltpu.sync_copy(mnb_v, omin_hbm.at[pl.ds(base, P)])

        pl.loop(0, n_chunks)(chunk_body)
        pltpu.sync_copy(ss_v, ss_hbm.at[wid])

    return k


def _finalize_body(omax_ref, omin_ref, ss_ref, alpha_ref, beta_ref, out_ref,
                   *, denom):
    sigma = jnp.sum(ss_ref[...]) / denom
    scale = 1.0 / (sigma + EPS)
    alpha = alpha_ref[...]
    sel = jnp.where(alpha >= 0.0, omax_ref[...], omin_ref[...])
    out_ref[...] = sel * (alpha * scale) + beta_ref[...]


def kernel(xyz_B3N, feats_BNC, alpha, beta):
    B, _, N = xyz_B3N.shape
    C = feats_BNC.shape[-1]
    BN = B * N

    xq = jnp.pad(jnp.swapaxes(xyz_B3N, 1, 2), ((0, 0), (0, 0), (0, 5)))
    xk = jnp.pad(xyz_B3N, ((0, 0), (0, 5), (0, 0)))
    d_sc, tau = _dist_tau_kernel(B, N, 1024)(xq, xk)

    gidx = _select_kernel(BN, N)(d_sc.reshape(BN * N), tau.reshape(BN))

    feats_flat = feats_BNC.reshape(BN, C)
    omax, omin, ss = _gather_reduce_kernel(BN, C, 4)(feats_flat, gidx)

    T = 2048
    out = pl.pallas_call(
        functools.partial(_finalize_body, denom=float(BN * K_NEIGH * C)),
        grid=(BN // T,),
        in_specs=[
            pl.BlockSpec((T, C), lambda i: (i, 0)),
            pl.BlockSpec((T, C), lambda i: (i, 0)),
            pl.BlockSpec((_NW, _L), lambda i: (0, 0)),
            pl.BlockSpec((1, C), lambda i: (0, 0)),
            pl.BlockSpec((1, C), lambda i: (0, 0)),
        ],
        out_specs=pl.BlockSpec((T, C), lambda i: (i, 0)),
        out_shape=jax.ShapeDtypeStruct((BN, C), jnp.float32),
    )(omax, omin, ss, alpha.reshape(1, C), beta.reshape(1, C))
    return out.reshape(B, N, C)


# speedup vs baseline: 1.2976x; 1.2976x over previous
import functools

import jax
import jax.numpy as jnp
from jax import lax
from jax.experimental import pallas as pl
from jax.experimental.pallas import tpu as pltpu
from jax.experimental.pallas import tpu_sc as plsc

K_NEIGH = 24
EPS = 1e-05

_NC = 2
_NS = 16
_NW = _NC * _NS
_L = 16
_NEG = float("-inf")


def _dist_tau_body(xq_ref, xk_ref, d_ref, tau_ref, *, R, N):
    xq = xq_ref[0]
    xk = xk_ref[0]
    mm = jnp.dot(xq, xk, preferred_element_type=jnp.float32)
    xxa = jnp.sum(xk * xk, axis=0, keepdims=True)
    xxr = jnp.sum(xq * xq, axis=1, keepdims=True)
    d = (2.0 * mm - xxa) - xxr

    cm = jnp.maximum(d[:, :N // 2], d[:, N // 2:])
    w = N // 2
    while w > 128:
        cm = jnp.maximum(cm[:, :w // 2], cm[:, w // 2:])
        w //= 2

    def rnd(_, carry):
        cmc, _m = carry
        m = jnp.max(cmc, axis=1, keepdims=True)
        cmc = jnp.where(cmc == m, _NEG, cmc)
        return cmc, m

    _, tau = lax.fori_loop(0, K_NEIGH, rnd, (cm, cm[:, :1]))

    d_ref[...] = d.reshape(R * 16, 128)
    tau_ref[...] = tau.reshape(R // 128, 128)


def _dist_tau_kernel(B, N, R):
    BN = B * N
    nr = N // R
    return pl.pallas_call(
        functools.partial(_dist_tau_body, R=R, N=N),
        grid=(B, nr),
        in_specs=[
            pl.BlockSpec((1, R, 8), lambda b, r: (b, r, 0)),
            pl.BlockSpec((1, 8, N), lambda b, r: (b, 0, 0)),
        ],
        out_specs=[
            pl.BlockSpec((R * 16, 128), lambda b, r: (b * nr + r, 0)),
            pl.BlockSpec((R // 128, 128), lambda b, r: (b * nr + r, 0)),
        ],
        out_shape=[
            jax.ShapeDtypeStruct((BN * 16, 128), jnp.float32),
            jax.ShapeDtypeStruct((BN // 128, 128), jnp.float32),
        ],
    )


def _dyn_gather16(vec, idx):
    return lax.gather(
        vec, idx[:, None],
        dimension_numbers=lax.GatherDimensionNumbers(
            offset_dims=(), collapsed_slice_dims=(0,), start_index_map=(0,)),
        slice_sizes=(1,),
        mode=lax.GatherScatterMode.PROMISE_IN_BOUNDS)


def _sort16kv(k, v):
    return plsc.sort_key_val(k, v, descending=True)


def _merge2kv(ka, va, kb, vb):
    rkb = lax.rev(kb, (0,))
    rvb = lax.rev(vb, (0,))
    sel = ka >= rkb
    hk, hv = _sort16kv(jnp.where(sel, ka, rkb), jnp.where(sel, va, rvb))
    lk, lv = _sort16kv(jnp.where(sel, rkb, ka), jnp.where(sel, rvb, va))
    return hk, hv, lk, lv


def _select_kernel(BN, N):
    RPW = BN // _NW
    RB = 8
    NB = RPW // RB
    NCHK = N // _L
    SROW = N // 128
    mesh = plsc.VectorSubcoreMesh(core_axis_name="c", subcore_axis_name="s")

    @functools.partial(
        pl.kernel,
        mesh=mesh,
        compiler_params=pltpu.CompilerParams(needs_layout_passes=False),
        out_type=jax.ShapeDtypeStruct((BN * K_NEIGH,), jnp.int32),

        scratch_types=[
            pltpu.VMEM((RB * SROW * 128,), jnp.float32),
            pltpu.VMEM((RPW,), jnp.float32),
            pltpu.VMEM((N + _L,), jnp.int32),
            pltpu.VMEM((RB * K_NEIGH + 2 * _L,), jnp.int32),
        ],
    )
    def k(d_hbm, tau_hbm, idx_hbm, dbuf, tau_v, sidx, outv):
        wid = lax.axis_index("s") * _NC + lax.axis_index("c")
        wrow0 = wid * RPW
        pltpu.sync_copy(tau_hbm.at[pl.ds(wid * RPW, RPW)], tau_v)
        lane = lax.iota(jnp.int32, _L)

        def block_body(g):
            rowbase = wrow0 + g * RB
            pltpu.sync_copy(d_hbm.at[pl.ds(rowbase * SROW * 128,
                                           RB * SROW * 128)], dbuf)

            def row_body(j):
                lr = g * RB + j
                rg = wrow0 + lr
                bt = plsc.load_gather(tau_v,
                                      [jnp.full((_L,), lr, jnp.int32)])

                def sub_body(sb, off):
                    base_f = (j * SROW + sb) * 128
                    ivb = jnp.full((_L,), sb * 128, jnp.int32)
                    for kk in range(8):
                        v = dbuf[pl.ds(base_f + kk * _L, _L)]
                        m = v >= bt
                        iv = ivb + (kk * _L + lane)
                        plsc.store_compressed(sidx.at[pl.ds(off, _L)], iv,
                                              mask=m)
                        off = off + plsc.all_reduce_population_count(m)[0]
                    return off

                cnt = lax.fori_loop(0, SROW, sub_body, jnp.int32(0),
                                    unroll=2)
                sv = (cnt + (_L - 1)) // _L

                def gathered(j2):
                    si = sidx[pl.ds(j2 * _L, _L)]
                    valid = (j2 * _L + lane) < cnt
                    x = plsc.load_gather(
                        dbuf, [si + jnp.full((_L,), j * N, jnp.int32)],
                        mask=valid)
                    return si, jnp.where(valid, x, _NEG)

                s0, x0 = gathered(0)
                k1, i1 = _sort16kv(x0, s0)
                s1, x1 = gathered(1)
                k2, i2 = _sort16kv(x1, s1)

                def tour_body(j2, T):
                    t1k, t1v, t2k, t2v = T
                    sj, xj = gathered(j2)
                    xs, xi = _sort16kv(xj, sj)
                    h1k, h1v, l1k, l1v = _merge2kv(t1k, t1v, xs, xi)
                    h2k, h2v, _lk, _lv = _merge2kv(t2k, t2v, l1k, l1v)
                    return h1k, h1v, h2k, h2v

                t1k, t1v, t2k, t2v = lax.fori_loop(
                    2, sv, tour_body, (k1, i1, k2, i2))
                _fk, f1v, _gk, f2v = _merge2kv(t1k, t1v, t2k, t2v)

                gbv = jnp.full((_L,), (rg // N) * N, jnp.int32)
                ebase = j * K_NEIGH
                outv[pl.ds(ebase, _L)] = f1v + gbv
                outv[pl.ds(ebase + _L, _L)] = f2v + gbv

            pl.loop(0, RB)(row_body)
            pltpu.sync_copy(
                outv.at[pl.ds(0, RB * K_NEIGH)],
                idx_hbm.at[pl.ds(rowbase * K_NEIGH, RB * K_NEIGH)])

        pl.loop(0, NB)(block_body)

    return k


def _gather_reduce_kernel(BN, C, P):
    pts_per_w = BN // _NW
    n_chunks = pts_per_w // P
    CV = C // _L
    mesh = plsc.VectorSubcoreMesh(core_axis_name="c", subcore_axis_name="s")

    @functools.partial(
        pl.kernel,
        mesh=mesh,
        compiler_params=pltpu.CompilerParams(needs_layout_passes=False),
        out_type=[
            jax.ShapeDtypeStruct((BN, C), jnp.float32),
            jax.ShapeDtypeStruct((BN, C), jnp.float32),
            jax.ShapeDtypeStruct((_NW, _L), jnp.float32),
        ],
        scratch_types=[
            pltpu.VMEM((P * K_NEIGH,), jnp.int32),
            pltpu.VMEM((P * K_NEIGH,), jnp.int32),
            pltpu.VMEM((P * K_NEIGH, C), jnp.float32),
            pltpu.VMEM((P * K_NEIGH, C), jnp.float32),
            pltpu.VMEM((P, C), jnp.float32),
            pltpu.VMEM((P, C), jnp.float32),
            pltpu.VMEM((P, C), jnp.float32),
            pltpu.VMEM((_L,), jnp.float32),
            pltpu.SemaphoreType.DMA,
            pltpu.SemaphoreType.DMA,
        ],
    )
    def k(feats_hbm, gidx_hbm, omax_hbm, omin_hbm, ss_hbm,
          idx_v0, idx_v1, rows_v0, rows_v1, cen_v, mxb_v, mnb_v, ss_v,
          sem0, sem1):
        wid = lax.axis_index("s") * _NC + lax.axis_index("c")
        w_base = wid * pts_per_w
        ss_v[...] = jnp.zeros((_L,), jnp.float32)

        def start(t, idx_v, rows_v, sem):
            base = w_base + t * P
            pltpu.sync_copy(gidx_hbm.at[pl.ds(base * K_NEIGH, P * K_NEIGH)],
                            idx_v)
            pltpu.async_copy(feats_hbm.at[idx_v], rows_v, sem)

        def compute(t, idx_v, rows_v, sem):
            base = w_base + t * P
            pltpu.make_async_copy(feats_hbm.at[idx_v], rows_v, sem).wait()
            pltpu.sync_copy(feats_hbm.at[pl.ds(base, P)], cen_v)

            def pt_body(p):
                ssa = ss_v[...]
                for c8 in range(CV):
                    cen = cen_v[p, pl.ds(c8 * _L, _L)]
                    d0 = rows_v[p * K_NEIGH, pl.ds(c8 * _L, _L)] - cen
                    mx = d0
                    mn = d0
                    sq = d0 * d0
                    for kk in range(1, K_NEIGH):
                        d = rows_v[p * K_NEIGH + kk, pl.ds(c8 * _L, _L)] - cen
                        mx = jnp.maximum(mx, d)
                        mn = jnp.minimum(mn, d)
                        sq = sq + d * d
                    mxb_v[p, pl.ds(c8 * _L, _L)] = mx
                    mnb_v[p, pl.ds(c8 * _L, _L)] = mn
                    ssa = ssa + sq
                ss_v[...] = ssa

            pl.loop(0, P)(pt_body)
            pltpu.sync_copy(mxb_v, omax_hbm.at[pl.ds(base, P)])
            pltpu.sync_copy(mnb_v, omin_hbm.at[pl.ds(base, P)])

        start(0, idx_v0, rows_v0, sem0)

        def pair_body(t2):
            t = t2 * 2
            start(t + 1, idx_v1, rows_v1, sem1)
            compute(t, idx_v0, rows_v0, sem0)

            @pl.when(t + 2 < n_chunks)
            def _():
                start(t + 2, idx_v0, rows_v0, sem0)

            compute(t + 1, idx_v1, rows_v1, sem1)

        pl.loop(0, n_chunks // 2)(pair_body)
        pltpu.sync_copy(ss_v, ss_hbm.at[wid])

    return k


def _finalize_body(omax_ref, omin_ref, ss_ref, alpha_ref, beta_ref, out_ref,
                   *, denom):
    sigma = jnp.sum(ss_ref[...]) / denom
    scale = 1.0 / (sigma + EPS)
    alpha = alpha_ref[...]
    sel = jnp.where(alpha >= 0.0, omax_ref[...], omin_ref[...])
    out_ref[...] = sel * (alpha * scale) + beta_ref[...]


def kernel(xyz_B3N, feats_BNC, alpha, beta):
    B, _, N = xyz_B3N.shape
    C = feats_BNC.shape[-1]
    BN = B * N

    xq = jnp.pad(jnp.swapaxes(xyz_B3N, 1, 2), ((0, 0), (0, 0), (0, 5)))
    xk = jnp.pad(xyz_B3N, ((0, 0), (0, 5), (0, 0)))
    d_sc, tau = _dist_tau_kernel(B, N, 1024)(xq, xk)

    gidx = _select_kernel(BN, N)(d_sc.reshape(BN * N), tau.reshape(BN))

    feats_flat = feats_BNC.reshape(BN, C)
    omax, omin, ss = _gather_reduce_kernel(BN, C, 4)(feats_flat, gidx)

    T = 2048
    out = pl.pallas_call(
        functools.partial(_finalize_body, denom=float(BN * K_NEIGH * C)),
        grid=(BN // T,),
        in_specs=[
            pl.BlockSpec((T, C), lambda i: (i, 0)),
            pl.BlockSpec((T, C), lambda i: (i, 0)),
            pl.BlockSpec((_NW, _L), lambda i: (0, 0)),
            pl.BlockSpec((1, C), lambda i: (0, 0)),
            pl.BlockSpec((1, C), lambda i: (0, 0)),
        ],
        out_specs=pl.BlockSpec((T, C), lambda i: (i, 0)),
        out_shape=jax.ShapeDtypeStruct((BN, C), jnp.float32),
    )(omax, omin, ss, alpha.reshape(1, C), beta.reshape(1, C))
    return out.reshape(B, N, C)


# speedup vs baseline: 1.5879x; 1.2237x over previous
import functools

import jax
import jax.numpy as jnp
from jax import lax
from jax.experimental import pallas as pl
from jax.experimental.pallas import tpu as pltpu
from jax.experimental.pallas import tpu_sc as plsc

K_NEIGH = 24
EPS = 1e-05

_NC = 2
_NS = 16
_NW = _NC * _NS
_L = 16
_NEG = float("-inf")


def _dist_tau_body(xq_ref, xk_ref, d_ref, tau_ref, *, R, N):
    xq = xq_ref[0]
    xk = xk_ref[0]
    mm = jnp.dot(xq, xk, preferred_element_type=jnp.float32)
    xxa = jnp.sum(xk * xk, axis=0, keepdims=True)
    xxr = jnp.sum(xq * xq, axis=1, keepdims=True)
    d = (2.0 * mm - xxa) - xxr

    cm = jnp.maximum(d[:, :N // 2], d[:, N // 2:])
    w = N // 2
    while w > 128:
        cm = jnp.maximum(cm[:, :w // 2], cm[:, w // 2:])
        w //= 2

    def rnd(_, carry):
        cmc, _m = carry
        m = jnp.max(cmc, axis=1, keepdims=True)
        cmc = jnp.where(cmc == m, _NEG, cmc)
        return cmc, m

    _, tau = lax.fori_loop(0, K_NEIGH, rnd, (cm, cm[:, :1]))

    d_ref[...] = d.reshape(R * 16, 128)
    tau_ref[...] = tau.reshape(R // 128, 128)


def _dist_tau_kernel(B, N, R):
    BN = B * N
    nr = N // R
    return pl.pallas_call(
        functools.partial(_dist_tau_body, R=R, N=N),
        grid=(B, nr),
        in_specs=[
            pl.BlockSpec((1, R, 8), lambda b, r: (b, r, 0)),
            pl.BlockSpec((1, 8, N), lambda b, r: (b, 0, 0)),
        ],
        out_specs=[
            pl.BlockSpec((R * 16, 128), lambda b, r: (b * nr + r, 0)),
            pl.BlockSpec((R // 128, 128), lambda b, r: (b * nr + r, 0)),
        ],
        out_shape=[
            jax.ShapeDtypeStruct((BN * 16, 128), jnp.float32),
            jax.ShapeDtypeStruct((BN // 128, 128), jnp.float32),
        ],
    )


def _dyn_gather16(vec, idx):
    return lax.gather(
        vec, idx[:, None],
        dimension_numbers=lax.GatherDimensionNumbers(
            offset_dims=(), collapsed_slice_dims=(0,), start_index_map=(0,)),
        slice_sizes=(1,),
        mode=lax.GatherScatterMode.PROMISE_IN_BOUNDS)


def _sort16kv(k, v):
    return plsc.sort_key_val(k, v, descending=True)


def _merge2kv(ka, va, kb, vb):
    rkb = lax.rev(kb, (0,))
    rvb = lax.rev(vb, (0,))
    sel = ka >= rkb
    hk, hv = _sort16kv(jnp.where(sel, ka, rkb), jnp.where(sel, va, rvb))
    lk, lv = _sort16kv(jnp.where(sel, rkb, ka), jnp.where(sel, rvb, va))
    return hk, hv, lk, lv


def _select_kernel(BN, N):
    RPW = BN // _NW
    RB = 8
    NB = RPW // RB
    NCHK = N // _L
    SROW = N // 128
    mesh = plsc.VectorSubcoreMesh(core_axis_name="c", subcore_axis_name="s")

    @functools.partial(
        pl.kernel,
        mesh=mesh,
        compiler_params=pltpu.CompilerParams(needs_layout_passes=False),
        out_type=jax.ShapeDtypeStruct((BN * K_NEIGH,), jnp.int32),

        scratch_types=[
            pltpu.VMEM((RB * SROW * 128,), jnp.float32),
            pltpu.VMEM((RPW,), jnp.float32),
            pltpu.VMEM((N + _L,), jnp.int32),
            pltpu.VMEM((N + _L,), jnp.int32),
            pltpu.VMEM((RB * K_NEIGH + 2 * _L,), jnp.int32),
        ],
    )
    def k(d_hbm, tau_hbm, idx_hbm, dbuf, tau_v, sidx, sidxB, outv):
        wid = lax.axis_index("s") * _NC + lax.axis_index("c")
        wrow0 = wid * RPW
        pltpu.sync_copy(tau_hbm.at[pl.ds(wid * RPW, RPW)], tau_v)
        lane = lax.iota(jnp.int32, _L)

        def block_body(g):
            rowbase = wrow0 + g * RB
            pltpu.sync_copy(d_hbm.at[pl.ds(rowbase * SROW * 128,
                                           RB * SROW * 128)], dbuf)

            def row_body(jj):
                j = jj * 2
                jB = j + 1
                lr = g * RB + j
                rg = wrow0 + lr
                bt = plsc.load_gather(tau_v,
                                      [jnp.full((_L,), lr, jnp.int32)])
                btB = plsc.load_gather(tau_v,
                                       [jnp.full((_L,), lr + 1, jnp.int32)])

                def sub_body(sb, offs):
                    off, offB = offs
                    base_f = (j * SROW + sb) * 128
                    base_fB = (jB * SROW + sb) * 128
                    ivb = jnp.full((_L,), sb * 128, jnp.int32)
                    for kk in range(8):
                        iv = ivb + (kk * _L + lane)
                        v = dbuf[pl.ds(base_f + kk * _L, _L)]
                        vB = dbuf[pl.ds(base_fB + kk * _L, _L)]
                        m = v >= bt
                        mB = vB >= btB
                        plsc.store_compressed(sidx.at[pl.ds(off, _L)], iv,
                                              mask=m)
                        plsc.store_compressed(sidxB.at[pl.ds(offB, _L)], iv,
                                              mask=mB)
                        off = off + plsc.all_reduce_population_count(m)[0]
                        offB = offB + plsc.all_reduce_population_count(mB)[0]
                    return off, offB

                cnt, cntB = lax.fori_loop(0, SROW, sub_body,
                                          (jnp.int32(0), jnp.int32(0)))
                sv = (cnt + (_L - 1)) // _L
                svB = (cntB + (_L - 1)) // _L

                def gathered(sx, jr, cn, j2):
                    si = sx[pl.ds(j2 * _L, _L)]
                    valid = (j2 * _L + lane) < cn
                    x = plsc.load_gather(
                        dbuf, [si + jnp.full((_L,), jr * N, jnp.int32)],
                        mask=valid)
                    return si, jnp.where(valid, x, _NEG)

                s0, x0 = gathered(sidx, j, cnt, 0)
                s0B, x0B = gathered(sidxB, jB, cntB, 0)
                k1, i1 = _sort16kv(x0, s0)
                k1B, i1B = _sort16kv(x0B, s0B)
                s1, x1 = gathered(sidx, j, cnt, 1)
                s1B, x1B = gathered(sidxB, jB, cntB, 1)
                k2, i2 = _sort16kv(x1, s1)
                k2B, i2B = _sort16kv(x1B, s1B)

                def tour_body(sx, jr, cn):
                    def body(j2, T):
                        t1k, t1v, t2k, t2v = T
                        sj, xj = gathered(sx, jr, cn, j2)
                        xs, xi = _sort16kv(xj, sj)
                        h1k, h1v, l1k, l1v = _merge2kv(t1k, t1v, xs, xi)
                        h2k, h2v, _lk, _lv = _merge2kv(t2k, t2v, l1k, l1v)
                        return h1k, h1v, h2k, h2v
                    return body

                t1k, t1v, t2k, t2v = lax.fori_loop(
                    2, sv, tour_body(sidx, j, cnt), (k1, i1, k2, i2))
                t1kB, t1vB, t2kB, t2vB = lax.fori_loop(
                    2, svB, tour_body(sidxB, jB, cntB), (k1B, i1B, k2B, i2B))
                _fk, f1v, _gk, f2v = _merge2kv(t1k, t1v, t2k, t2v)
                _fkB, f1vB, _gkB, f2vB = _merge2kv(t1kB, t1vB, t2kB, t2vB)

                gbv = jnp.full((_L,), (rg // N) * N, jnp.int32)
                ebase = j * K_NEIGH
                outv[pl.ds(ebase, _L)] = f1v + gbv
                outv[pl.ds(ebase + _L, _L)] = f2v + gbv
                ebaseB = jB * K_NEIGH
                outv[pl.ds(ebaseB, _L)] = f1vB + gbv
                outv[pl.ds(ebaseB + _L, _L)] = f2vB + gbv

            pl.loop(0, RB // 2)(row_body)
            pltpu.sync_copy(
                outv.at[pl.ds(0, RB * K_NEIGH)],
                idx_hbm.at[pl.ds(rowbase * K_NEIGH, RB * K_NEIGH)])

        pl.loop(0, NB)(block_body)

    return k


def _gather_reduce_kernel(BN, C, P):
    pts_per_w = BN // _NW
    n_chunks = pts_per_w // P
    CV = C // _L
    mesh = plsc.VectorSubcoreMesh(core_axis_name="c", subcore_axis_name="s")

    @functools.partial(
        pl.kernel,
        mesh=mesh,
        compiler_params=pltpu.CompilerParams(needs_layout_passes=False),
        out_type=[
            jax.ShapeDtypeStruct((BN, C), jnp.float32),
            jax.ShapeDtypeStruct((BN, C), jnp.float32),
            jax.ShapeDtypeStruct((_NW, _L), jnp.float32),
        ],
        scratch_types=[
            pltpu.VMEM((P * K_NEIGH,), jnp.int32),
            pltpu.VMEM((P * K_NEIGH,), jnp.int32),
            pltpu.VMEM((P * K_NEIGH, C), jnp.float32),
            pltpu.VMEM((P * K_NEIGH, C), jnp.float32),
            pltpu.VMEM((P, C), jnp.float32),
            pltpu.VMEM((P, C), jnp.float32),
            pltpu.VMEM((P, C), jnp.float32),
            pltpu.VMEM((_L,), jnp.float32),
            pltpu.SemaphoreType.DMA,
            pltpu.SemaphoreType.DMA,
        ],
    )
    def k(feats_hbm, gidx_hbm, omax_hbm, omin_hbm, ss_hbm,
          idx_v0, idx_v1, rows_v0, rows_v1, cen_v, mxb_v, mnb_v, ss_v,
          sem0, sem1):
        wid = lax.axis_index("s") * _NC + lax.axis_index("c")
        w_base = wid * pts_per_w
        ss_v[...] = jnp.zeros((_L,), jnp.float32)

        def start(t, idx_v, rows_v, sem):
            base = w_base + t * P
            pltpu.sync_copy(gidx_hbm.at[pl.ds(base * K_NEIGH, P * K_NEIGH)],
                            idx_v)
            pltpu.async_copy(feats_hbm.at[idx_v], rows_v, sem)

        def compute(t, idx_v, rows_v, sem):
            base = w_base + t * P
            pltpu.make_async_copy(feats_hbm.at[idx_v], rows_v, sem).wait()
            pltpu.sync_copy(feats_hbm.at[pl.ds(base, P)], cen_v)

            def pt_body(p):
                ssa = ss_v[...]
                for c8 in range(CV):
                    cen = cen_v[p, pl.ds(c8 * _L, _L)]
                    d0 = rows_v[p * K_NEIGH, pl.ds(c8 * _L, _L)] - cen
                    mx = d0
                    mn = d0
                    sq = d0 * d0
                    for kk in range(1, K_NEIGH):
                        d = rows_v[p * K_NEIGH + kk, pl.ds(c8 * _L, _L)] - cen
                        mx = jnp.maximum(mx, d)
                        mn = jnp.minimum(mn, d)
                        sq = sq + d * d
                    mxb_v[p, pl.ds(c8 * _L, _L)] = mx
                    mnb_v[p, pl.ds(c8 * _L, _L)] = mn
                    ssa = ssa + sq
                ss_v[...] = ssa

            pl.loop(0, P)(pt_body)
            pltpu.sync_copy(mxb_v, omax_hbm.at[pl.ds(base, P)])
            pltpu.sync_copy(mnb_v, omin_hbm.at[pl.ds(base, P)])

        start(0, idx_v0, rows_v0, sem0)

        def pair_body(t2):
            t = t2 * 2
            start(t + 1, idx_v1, rows_v1, sem1)
            compute(t, idx_v0, rows_v0, sem0)

            @pl.when(t + 2 < n_chunks)
            def _():
                start(t + 2, idx_v0, rows_v0, sem0)

            compute(t + 1, idx_v1, rows_v1, sem1)

        pl.loop(0, n_chunks // 2)(pair_body)
        pltpu.sync_copy(ss_v, ss_hbm.at[wid])

    return k


def _finalize_body(omax_ref, omin_ref, ss_ref, alpha_ref, beta_ref, out_ref,
                   *, denom):
    sigma = jnp.sum(ss_ref[...]) / denom
    scale = 1.0 / (sigma + EPS)
    alpha = alpha_ref[...]
    sel = jnp.where(alpha >= 0.0, omax_ref[...], omin_ref[...])
    out_ref[...] = sel * (alpha * scale) + beta_ref[...]


def kernel(xyz_B3N, feats_BNC, alpha, beta):
    B, _, N = xyz_B3N.shape
    C = feats_BNC.shape[-1]
    BN = B * N

    xq = jnp.pad(jnp.swapaxes(xyz_B3N, 1, 2), ((0, 0), (0, 0), (0, 5)))
    xk = jnp.pad(xyz_B3N, ((0, 0), (0, 5), (0, 0)))
    d_sc, tau = _dist_tau_kernel(B, N, 1024)(xq, xk)

    gidx = _select_kernel(BN, N)(d_sc.reshape(BN * N), tau.reshape(BN))

    feats_flat = feats_BNC.reshape(BN, C)
    omax, omin, ss = _gather_reduce_kernel(BN, C, 4)(feats_flat, gidx)

    T = 2048
    out = pl.pallas_call(
        functools.partial(_finalize_body, denom=float(BN * K_NEIGH * C)),
        grid=(BN // T,),
        in_specs=[
            pl.BlockSpec((T, C), lambda i: (i, 0)),
            pl.BlockSpec((T, C), lambda i: (i, 0)),
            pl.BlockSpec((_NW, _L), lambda i: (0, 0)),
            pl.BlockSpec((1, C), lambda i: (0, 0)),
            pl.BlockSpec((1, C), lambda i: (0, 0)),
        ],
        out_specs=pl.BlockSpec((T, C), lambda i: (i, 0)),
        out_shape=jax.ShapeDtypeStruct((BN, C), jnp.float32),
    )(omax, omin, ss, alpha.reshape(1, C), beta.reshape(1, C))
    return out.reshape(B, N, C)


# speedup vs baseline: 1.7341x; 1.0921x over previous
import functools

import jax
import jax.numpy as jnp
from jax import lax
from jax.experimental import pallas as pl
from jax.experimental.pallas import tpu as pltpu
from jax.experimental.pallas import tpu_sc as plsc

K_NEIGH = 24
EPS = 1e-05

_NC = 2
_NS = 16
_NW = _NC * _NS
_L = 16
_NEG = float("-inf")


def _dist_tau_body(xq_ref, xk_ref, d_ref, tau_ref, *, R, N):
    xq = xq_ref[0]
    xk = xk_ref[0]
    mm = jnp.dot(xq, xk, preferred_element_type=jnp.float32)
    xxa = jnp.sum(xk * xk, axis=0, keepdims=True)
    xxr = jnp.sum(xq * xq, axis=1, keepdims=True)
    d = (2.0 * mm - xxa) - xxr

    cm = jnp.maximum(d[:, :N // 2], d[:, N // 2:])
    w = N // 2
    while w > 128:
        cm = jnp.maximum(cm[:, :w // 2], cm[:, w // 2:])
        w //= 2

    def rnd(_, carry):
        cmc, _m = carry
        m = jnp.max(cmc, axis=1, keepdims=True)
        cmc = jnp.where(cmc == m, _NEG, cmc)
        return cmc, m

    _, tau = lax.fori_loop(0, K_NEIGH, rnd, (cm, cm[:, :1]))

    d_ref[...] = d.reshape(R * 16, 128)
    tau_ref[...] = tau.reshape(R // 128, 128)


def _dist_tau_kernel(B, N, R):
    BN = B * N
    nr = N // R
    return pl.pallas_call(
        functools.partial(_dist_tau_body, R=R, N=N),
        grid=(B, nr),
        in_specs=[
            pl.BlockSpec((1, R, 8), lambda b, r: (b, r, 0)),
            pl.BlockSpec((1, 8, N), lambda b, r: (b, 0, 0)),
        ],
        out_specs=[
            pl.BlockSpec((R * 16, 128), lambda b, r: (b * nr + r, 0)),
            pl.BlockSpec((R // 128, 128), lambda b, r: (b * nr + r, 0)),
        ],
        out_shape=[
            jax.ShapeDtypeStruct((BN * 16, 128), jnp.float32),
            jax.ShapeDtypeStruct((BN // 128, 128), jnp.float32),
        ],
    )


def _dyn_gather16(vec, idx):
    return lax.gather(
        vec, idx[:, None],
        dimension_numbers=lax.GatherDimensionNumbers(
            offset_dims=(), collapsed_slice_dims=(0,), start_index_map=(0,)),
        slice_sizes=(1,),
        mode=lax.GatherScatterMode.PROMISE_IN_BOUNDS)


def _sort16kv(k, v):
    return plsc.sort_key_val(k, v, descending=True)


def _merge2kv(ka, va, kb, vb):
    rkb = lax.rev(kb, (0,))
    rvb = lax.rev(vb, (0,))
    sel = ka >= rkb
    hk, hv = _sort16kv(jnp.where(sel, ka, rkb), jnp.where(sel, va, rvb))
    lk, lv = _sort16kv(jnp.where(sel, rkb, ka), jnp.where(sel, rvb, va))
    return hk, hv, lk, lv


def _select_kernel(BN, N):
    RPW = BN // _NW
    RB = 8
    NB = RPW // RB
    NCHK = N // _L
    SROW = N // 128
    mesh = plsc.VectorSubcoreMesh(core_axis_name="c", subcore_axis_name="s")

    @functools.partial(
        pl.kernel,
        mesh=mesh,
        compiler_params=pltpu.CompilerParams(needs_layout_passes=False),
        out_type=jax.ShapeDtypeStruct((BN * K_NEIGH,), jnp.int32),

        scratch_types=[
            pltpu.VMEM((RB * SROW * 128,), jnp.float32),
            pltpu.VMEM((RB * SROW * 128,), jnp.float32),
            pltpu.VMEM((RPW,), jnp.float32),
            pltpu.VMEM((N + _L,), jnp.int32),
            pltpu.VMEM((N + _L,), jnp.int32),
            pltpu.VMEM((RB * K_NEIGH + 2 * _L,), jnp.int32),
            pltpu.SemaphoreType.DMA,
            pltpu.SemaphoreType.DMA,
        ],
    )
    def k(d_hbm, tau_hbm, idx_hbm, dbufA, dbufB, tau_v, sidx, sidxB,
          outv, dsemA, dsemB):
        wid = lax.axis_index("s") * _NC + lax.axis_index("c")
        wrow0 = wid * RPW
        pltpu.sync_copy(tau_hbm.at[pl.ds(wid * RPW, RPW)], tau_v)
        lane = lax.iota(jnp.int32, _L)

        def dstart(g, dbuf, dsem):
            rowbase = wrow0 + g * RB
            pltpu.async_copy(
                d_hbm.at[pl.ds(rowbase * SROW * 128, RB * SROW * 128)],
                dbuf, dsem)

        def dwait(g, dbuf, dsem):
            rowbase = wrow0 + g * RB
            pltpu.make_async_copy(
                d_hbm.at[pl.ds(rowbase * SROW * 128, RB * SROW * 128)],
                dbuf, dsem).wait()

        def block_body(g, dbuf):
            rowbase = wrow0 + g * RB

            def row_body(jj):
                j = jj * 2
                jB = j + 1
                lr = g * RB + j
                rg = wrow0 + lr
                bt = plsc.load_gather(tau_v,
                                      [jnp.full((_L,), lr, jnp.int32)])
                btB = plsc.load_gather(tau_v,
                                       [jnp.full((_L,), lr + 1, jnp.int32)])

                def sub_body(sb, offs):
                    off, offB = offs
                    base_f = (j * SROW + sb) * 128
                    base_fB = (jB * SROW + sb) * 128
                    ivb = jnp.full((_L,), sb * 128, jnp.int32)
                    for kk in range(8):
                        iv = ivb + (kk * _L + lane)
                        v = dbuf[pl.ds(base_f + kk * _L, _L)]
                        vB = dbuf[pl.ds(base_fB + kk * _L, _L)]
                        m = v >= bt
                        mB = vB >= btB
                        plsc.store_compressed(sidx.at[pl.ds(off, _L)], iv,
                                              mask=m)
                        plsc.store_compressed(sidxB.at[pl.ds(offB, _L)], iv,
                                              mask=mB)
                        off = off + plsc.all_reduce_population_count(m)[0]
                        offB = offB + plsc.all_reduce_population_count(mB)[0]
                    return off, offB

                cnt, cntB = lax.fori_loop(0, SROW, sub_body,
                                          (jnp.int32(0), jnp.int32(0)))
                sv = (cnt + (_L - 1)) // _L
                svB = (cntB + (_L - 1)) // _L

                def gathered(sx, jr, cn, j2):
                    si = sx[pl.ds(j2 * _L, _L)]
                    valid = (j2 * _L + lane) < cn
                    x = plsc.load_gather(
                        dbuf, [si + jnp.full((_L,), jr * N, jnp.int32)],
                        mask=valid)
                    return si, jnp.where(valid, x, _NEG)

                s0, x0 = gathered(sidx, j, cnt, 0)
                s0B, x0B = gathered(sidxB, jB, cntB, 0)
                k1, i1 = _sort16kv(x0, s0)
                k1B, i1B = _sort16kv(x0B, s0B)
                s1, x1 = gathered(sidx, j, cnt, 1)
                s1B, x1B = gathered(sidxB, jB, cntB, 1)
                k2, i2 = _sort16kv(x1, s1)
                k2B, i2B = _sort16kv(x1B, s1B)

                def tour_body(sx, jr, cn):
                    def body(j2, T):
                        t1k, t1v, t2k, t2v = T
                        sj, xj = gathered(sx, jr, cn, j2)
                        xs, xi = _sort16kv(xj, sj)
                        h1k, h1v, l1k, l1v = _merge2kv(t1k, t1v, xs, xi)
                        h2k, h2v, _lk, _lv = _merge2kv(t2k, t2v, l1k, l1v)
                        return h1k, h1v, h2k, h2v
                    return body

                t1k, t1v, t2k, t2v = lax.fori_loop(
                    2, sv, tour_body(sidx, j, cnt), (k1, i1, k2, i2))
                t1kB, t1vB, t2kB, t2vB = lax.fori_loop(
                    2, svB, tour_body(sidxB, jB, cntB), (k1B, i1B, k2B, i2B))
                _fk, f1v, _gk, f2v = _merge2kv(t1k, t1v, t2k, t2v)
                _fkB, f1vB, _gkB, f2vB = _merge2kv(t1kB, t1vB, t2kB, t2vB)

                gbv = jnp.full((_L,), (rg // N) * N, jnp.int32)
                ebase = j * K_NEIGH
                outv[pl.ds(ebase, _L)] = f1v + gbv
                outv[pl.ds(ebase + _L, _L)] = f2v + gbv
                ebaseB = jB * K_NEIGH
                outv[pl.ds(ebaseB, _L)] = f1vB + gbv
                outv[pl.ds(ebaseB + _L, _L)] = f2vB + gbv

            pl.loop(0, RB // 2)(row_body)
            pltpu.sync_copy(
                outv.at[pl.ds(0, RB * K_NEIGH)],
                idx_hbm.at[pl.ds(rowbase * K_NEIGH, RB * K_NEIGH)])

        dstart(0, dbufA, dsemA)

        def pair_blocks(g2):
            g = g2 * 2
            dstart(g + 1, dbufB, dsemB)
            dwait(g, dbufA, dsemA)
            block_body(g, dbufA)

            @pl.when(g + 2 < NB)
            def _():
                dstart(g + 2, dbufA, dsemA)

            dwait(g + 1, dbufB, dsemB)
            block_body(g + 1, dbufB)

        pl.loop(0, NB // 2)(pair_blocks)

    return k


def _gather_reduce_kernel(BN, C, P):
    pts_per_w = BN // _NW
    n_chunks = pts_per_w // P
    CV = C // _L
    mesh = plsc.VectorSubcoreMesh(core_axis_name="c", subcore_axis_name="s")

    @functools.partial(
        pl.kernel,
        mesh=mesh,
        compiler_params=pltpu.CompilerParams(needs_layout_passes=False),
        out_type=[
            jax.ShapeDtypeStruct((BN, C), jnp.float32),
            jax.ShapeDtypeStruct((BN, C), jnp.float32),
            jax.ShapeDtypeStruct((_NW, _L), jnp.float32),
        ],
        scratch_types=[
            pltpu.VMEM((P * K_NEIGH,), jnp.int32),
            pltpu.VMEM((P * K_NEIGH,), jnp.int32),
            pltpu.VMEM((P * K_NEIGH, C), jnp.float32),
            pltpu.VMEM((P * K_NEIGH, C), jnp.float32),
            pltpu.VMEM((P, C), jnp.float32),
            pltpu.VMEM((P, C), jnp.float32),
            pltpu.VMEM((P, C), jnp.float32),
            pltpu.VMEM((_L,), jnp.float32),
            pltpu.SemaphoreType.DMA,
            pltpu.SemaphoreType.DMA,
        ],
    )
    def k(feats_hbm, gidx_hbm, omax_hbm, omin_hbm, ss_hbm,
          idx_v0, idx_v1, rows_v0, rows_v1, cen_v, mxb_v, mnb_v, ss_v,
          sem0, sem1):
        wid = lax.axis_index("s") * _NC + lax.axis_index("c")
        w_base = wid * pts_per_w
        ss_v[...] = jnp.zeros((_L,), jnp.float32)

        def start(t, idx_v, rows_v, sem):
            base = w_base + t * P
            pltpu.sync_copy(gidx_hbm.at[pl.ds(base * K_NEIGH, P * K_NEIGH)],
                            idx_v)
            pltpu.async_copy(feats_hbm.at[idx_v], rows_v, sem)

        def compute(t, idx_v, rows_v, sem):
            base = w_base + t * P
            pltpu.make_async_copy(feats_hbm.at[idx_v], rows_v, sem).wait()
            pltpu.sync_copy(feats_hbm.at[pl.ds(base, P)], cen_v)

            def pt_body(p):
                ssa = ss_v[...]
                for c8 in range(CV):
                    cen = cen_v[p, pl.ds(c8 * _L, _L)]
                    d0 = rows_v[p * K_NEIGH, pl.ds(c8 * _L, _L)] - cen
                    mx = d0
                    mn = d0
                    sq = d0 * d0
                    for kk in range(1, K_NEIGH):
                        d = rows_v[p * K_NEIGH + kk, pl.ds(c8 * _L, _L)] - cen
                        mx = jnp.maximum(mx, d)
                        mn = jnp.minimum(mn, d)
                        sq = sq + d * d
                    mxb_v[p, pl.ds(c8 * _L, _L)] = mx
                    mnb_v[p, pl.ds(c8 * _L, _L)] = mn
                    ssa = ssa + sq
                ss_v[...] = ssa

            pl.loop(0, P)(pt_body)
            pltpu.sync_copy(mxb_v, omax_hbm.at[pl.ds(base, P)])
            pltpu.sync_copy(mnb_v, omin_hbm.at[pl.ds(base, P)])

        start(0, idx_v0, rows_v0, sem0)

        def pair_body(t2):
            t = t2 * 2
            start(t + 1, idx_v1, rows_v1, sem1)
            compute(t, idx_v0, rows_v0, sem0)

            @pl.when(t + 2 < n_chunks)
            def _():
                start(t + 2, idx_v0, rows_v0, sem0)

            compute(t + 1, idx_v1, rows_v1, sem1)

        pl.loop(0, n_chunks // 2)(pair_body)
        pltpu.sync_copy(ss_v, ss_hbm.at[wid])

    return k


def _finalize_body(omax_ref, omin_ref, ss_ref, alpha_ref, beta_ref, out_ref,
                   *, denom):
    sigma = jnp.sum(ss_ref[...]) / denom
    scale = 1.0 / (sigma + EPS)
    alpha = alpha_ref[...]
    sel = jnp.where(alpha >= 0.0, omax_ref[...], omin_ref[...])
    out_ref[...] = sel * (alpha * scale) + beta_ref[...]


def kernel(xyz_B3N, feats_BNC, alpha, beta):
    B, _, N = xyz_B3N.shape
    C = feats_BNC.shape[-1]
    BN = B * N

    xq = jnp.pad(jnp.swapaxes(xyz_B3N, 1, 2), ((0, 0), (0, 0), (0, 5)))
    xk = jnp.pad(xyz_B3N, ((0, 0), (0, 5), (0, 0)))
    d_sc, tau = _dist_tau_kernel(B, N, 1024)(xq, xk)

    gidx = _select_kernel(BN, N)(d_sc.reshape(BN * N), tau.reshape(BN))

    feats_flat = feats_BNC.reshape(BN, C)
    omax, omin, ss = _gather_reduce_kernel(BN, C, 4)(feats_flat, gidx)

    T = 2048
    out = pl.pallas_call(
        functools.partial(_finalize_body, denom=float(BN * K_NEIGH * C)),
        grid=(BN // T,),
        in_specs=[
            pl.BlockSpec((T, C), lambda i: (i, 0)),
            pl.BlockSpec((T, C), lambda i: (i, 0)),
            pl.BlockSpec((_NW, _L), lambda i: (0, 0)),
            pl.BlockSpec((1, C), lambda i: (0, 0)),
            pl.BlockSpec((1, C), lambda i: (0, 0)),
        ],
        out_specs=pl.BlockSpec((T, C), lambda i: (i, 0)),
        out_shape=jax.ShapeDtypeStruct((BN, C), jnp.float32),
    )(omax, omin, ss, alpha.reshape(1, C), beta.reshape(1, C))
    return out.reshape(B, N, C)
